# Initial kernel scaffold; baseline (speedup 1.0000x reference)
import jax
import jax.numpy as jnp
from jax.experimental import pallas as pl


def kernel(features, edge_look_up, kernel):
    raise NotImplementedError("write your pallas kernel here")


# speedup vs baseline: 1.1063x; 1.1063x over previous
import functools

import jax
import jax.numpy as jnp
from jax import lax
from jax.experimental import pallas as pl
from jax.experimental.pallas import tpu as pltpu
from jax.experimental.pallas import tpu_sc as plsc

N = 10000
DEG = 32
D = 128
LANES = 16
NC, NS = 2, 16
NW = NC * NS
PW = 320
N_PAD = NW * PW
C = 4
EC = C * DEG
NCHUNK = PW // C


def _sc_gather_mean(features, idx_flat):
    mesh = plsc.VectorSubcoreMesh(core_axis_name="c", subcore_axis_name="s")

    @functools.partial(
        pl.kernel,
        mesh=mesh,
        out_type=jax.ShapeDtypeStruct((N_PAD, D), jnp.float32),
        scratch_types=[
            pltpu.VMEM((EC,), jnp.int32),
            pltpu.VMEM((EC, D), jnp.float32),
            pltpu.VMEM((C, D), jnp.float32),
            pltpu.SemaphoreType.DMA,
        ],
    )
    def k(feat_hbm, idx_hbm, out_hbm, idx_v, rows_v, out_v, sem):
        wid = lax.axis_index("s") * NC + lax.axis_index("c")
        base = wid * PW

        def chunk_body(ci, carry):
            node0 = base + ci * C
            pltpu.sync_copy(idx_hbm.at[pl.ds(node0 * DEG, EC)], idx_v)
            pltpu.async_copy(feat_hbm.at[idx_v], rows_v, sem).wait()
            for c in range(C):
                for g in range(D // LANES):
                    sl = pl.ds(g * LANES, LANES)
                    acc = rows_v[c * DEG, sl]
                    for r in range(1, DEG):
                        acc = acc + rows_v[c * DEG + r, sl]
                    out_v[c, sl] = acc * (1.0 / DEG)
            pltpu.sync_copy(out_v, out_hbm.at[pl.ds(node0, C)])
            return carry

        lax.fori_loop(0, NCHUNK, chunk_body, 0)

    return k(features, idx_flat)


def _tc_dense(features, agg, w_top, w_bot):
    bn = 2000

    def body(f_ref, a_ref, wt_ref, wb_ref, o_ref):
        acc = jnp.dot(f_ref[...], wt_ref[...], preferred_element_type=jnp.float32)
        acc = acc + jnp.dot(a_ref[...], wb_ref[...], preferred_element_type=jnp.float32)
        o_ref[...] = jnp.maximum(acc, 0.0)

    return pl.pallas_call(
        body,
        grid=(N // bn,),
        in_specs=[
            pl.BlockSpec((bn, D), lambda i: (i, 0)),
            pl.BlockSpec((bn, D), lambda i: (i, 0)),
            pl.BlockSpec((D, D), lambda i: (0, 0)),
            pl.BlockSpec((D, D), lambda i: (0, 0)),
        ],
        out_specs=pl.BlockSpec((bn, D), lambda i: (i, 0)),
        out_shape=jax.ShapeDtypeStruct((N, D), jnp.float32),
    )(features, agg, w_top, w_bot)


def kernel(features, edge_look_up, kernel):
    idx = edge_look_up.astype(jnp.int32).reshape(-1)
    idx = jnp.pad(idx, (0, N_PAD * DEG - idx.shape[0]))
    agg = _sc_gather_mean(features, idx)[:N]
    return _tc_dense(features, agg, kernel[:D], kernel[D:])


# speedup vs baseline: 1.3873x; 1.2540x over previous
import functools

import jax
import jax.numpy as jnp
from jax import lax
from jax.experimental import pallas as pl
from jax.experimental.pallas import tpu as pltpu
from jax.experimental.pallas import tpu_sc as plsc

N = 10000
DEG = 32
D = 128
LANES = 16
NC, NS = 2, 16
NW = NC * NS
PW = 320
N_PAD = NW * PW
C = 4
EC = C * DEG
NCHUNK = PW // C


def _sc_gather_mean(features, idx_chunks):
    mesh = plsc.VectorSubcoreMesh(core_axis_name="c", subcore_axis_name="s")

    @functools.partial(
        pl.kernel,
        mesh=mesh,
        out_type=jax.ShapeDtypeStruct((N_PAD, D), jnp.float32),
        scratch_types=[
            pltpu.VMEM((NCHUNK, EC), jnp.int32),
            pltpu.VMEM((EC, D), jnp.float32),
            pltpu.VMEM((EC, D), jnp.float32),
            pltpu.VMEM((PW, D), jnp.float32),
            pltpu.SemaphoreType.DMA,
            pltpu.SemaphoreType.DMA,
        ],
    )
    def k(feat_hbm, idx_hbm, out_hbm, idx_v, rows0, rows1, out_v, sem0, sem1):
        wid = lax.axis_index("s") * NC + lax.axis_index("c")
        base = wid * PW

        pltpu.sync_copy(idx_hbm.at[pl.ds(wid * NCHUNK, NCHUNK)], idx_v)

        def start(ci, rows, sem):
            pltpu.async_copy(feat_hbm.at[idx_v.at[ci]], rows, sem)

        def wait(ci, rows, sem):
            pltpu.make_async_copy(feat_hbm.at[idx_v.at[ci]], rows, sem).wait()

        def compute(rows, ci):
            for c in range(C):
                row_out = ci * C + c
                for g in range(D // LANES):
                    sl = pl.ds(g * LANES, LANES)
                    acc = rows[c * DEG, sl]
                    for r in range(1, DEG):
                        acc = acc + rows[c * DEG + r, sl]
                    out_v[row_out, sl] = acc * (1.0 / DEG)

        start(0, rows0, sem0)

        def pair_body(p, carry):
            c0 = 2 * p
            start(c0 + 1, rows1, sem1)
            wait(c0, rows0, sem0)
            compute(rows0, c0)
            start(jnp.minimum(c0 + 2, NCHUNK - 1), rows0, sem0)
            wait(c0 + 1, rows1, sem1)
            compute(rows1, c0 + 1)
            return carry

        lax.fori_loop(0, NCHUNK // 2, pair_body, 0)
        wait(NCHUNK - 1, rows0, sem0)
        pltpu.sync_copy(out_v, out_hbm.at[pl.ds(base, PW)])

    return k(features, idx_chunks)


def _tc_dense(features, agg, w_top, w_bot):
    bn = 2000

    def body(f_ref, a_ref, wt_ref, wb_ref, o_ref):
        acc = jnp.dot(f_ref[...], wt_ref[...], preferred_element_type=jnp.float32)
        acc = acc + jnp.dot(a_ref[...], wb_ref[...], preferred_element_type=jnp.float32)
        o_ref[...] = jnp.maximum(acc, 0.0)

    return pl.pallas_call(
        body,
        grid=(N // bn,),
        in_specs=[
            pl.BlockSpec((bn, D), lambda i: (i, 0)),
            pl.BlockSpec((bn, D), lambda i: (i, 0)),
            pl.BlockSpec((D, D), lambda i: (0, 0)),
            pl.BlockSpec((D, D), lambda i: (0, 0)),
        ],
        out_specs=pl.BlockSpec((bn, D), lambda i: (i, 0)),
        out_shape=jax.ShapeDtypeStruct((N, D), jnp.float32),
    )(features, agg, w_top, w_bot)


def kernel(features, edge_look_up, kernel):
    idx = edge_look_up.astype(jnp.int32).reshape(-1)
    idx = jnp.pad(idx, (0, N_PAD * DEG - idx.shape[0]))
    agg = _sc_gather_mean(features, idx.reshape(N_PAD // C, EC))[:N]
    return _tc_dense(features, agg, kernel[:D], kernel[D:])


# speedup vs baseline: 2.3631x; 1.7034x over previous
import functools

import jax
import jax.numpy as jnp
from jax import lax
from jax.experimental import pallas as pl
from jax.experimental.pallas import tpu as pltpu
from jax.experimental.pallas import tpu_sc as plsc

N = 10000
DEG = 32
D = 128
LANES = 16
NC, NS = 2, 16
NW = NC * NS
N_PAD = 10240
PW = N_PAD // NW
C = 4
EC = C * DEG
NCHUNK = PW // C
NPH = 2
CPH = NCHUNK // NPH
FCH = 20


def _sc_gather_mean(features, idx_chunks):
    mesh = plsc.VectorSubcoreMesh(core_axis_name="c", subcore_axis_name="s")

    @functools.partial(
        pl.kernel,
        mesh=mesh,
        out_type=jax.ShapeDtypeStruct((N_PAD, D), jnp.float32),
        scratch_types=[
            pltpu.VMEM_SHARED((N_PAD, D), jnp.float32),
            pltpu.VMEM((CPH, EC), jnp.int32),
            pltpu.VMEM((2 * EC, D), jnp.float32),
            pltpu.VMEM((FCH * C, D), jnp.float32),
            pltpu.SemaphoreType.DMA,
            pltpu.SemaphoreType.DMA,
        ],
    )
    def k(feat_hbm, idx_hbm, out_hbm, feat_sh, idx_v, rows_v, out_v,
          sem0, sem1):
        cid = lax.axis_index("c")
        sid = lax.axis_index("s")
        wid = sid * NC + cid
        base = wid * PW

        stripe = N_PAD // NS
        pltpu.sync_copy(feat_hbm.at[pl.ds(sid * stripe, stripe)],
                        feat_sh.at[pl.ds(sid * stripe, stripe)])
        plsc.subcore_barrier()

        def start(ci, half, sem):
            off = pl.multiple_of(half * EC, EC)
            pltpu.async_copy(feat_sh.at[idx_v.at[ci]],
                             rows_v.at[pl.ds(off, EC)], sem)

        def wait(ci, half, sem):
            off = pl.multiple_of(half * EC, EC)
            pltpu.make_async_copy(feat_sh.at[idx_v.at[ci]],
                                  rows_v.at[pl.ds(off, EC)], sem).wait()

        for ph in range(NPH):
            pltpu.sync_copy(
                idx_hbm.at[pl.ds(wid * NCHUNK + ph * CPH, CPH)], idx_v)
            start(0, 0, sem0)

            def chunk_body(ci, carry, ph=ph):
                half = lax.rem(ci, 2)
                nhalf = 1 - half
                nxt = jnp.minimum(ci + 1, CPH - 1)

                @pl.when(ci < CPH - 1)
                def _():
                    @pl.when(nhalf == 0)
                    def _():
                        start(nxt, 0, sem0)

                    @pl.when(nhalf == 1)
                    def _():
                        start(nxt, 1, sem1)

                @pl.when(half == 0)
                def _():
                    wait(ci, 0, sem0)

                @pl.when(half == 1)
                def _():
                    wait(ci, 1, sem1)

                rb = half * EC
                loc = lax.rem(ci, FCH)
                for c in range(C):
                    row_out = loc * C + c
                    for g in range(D // LANES):
                        sl = pl.ds(g * LANES, LANES)
                        acc = rows_v[rb + c * DEG, sl]
                        for r in range(1, DEG):
                            acc = acc + rows_v[rb + c * DEG + r, sl]
                        out_v[row_out, sl] = acc * (1.0 / DEG)

                @pl.when(loc == FCH - 1)
                def _():
                    node0 = pl.multiple_of(
                        base + ph * CPH * C + (ci - (FCH - 1)) * C, FCH * C)
                    pltpu.sync_copy(out_v, out_hbm.at[pl.ds(node0, FCH * C)])

                return carry

            lax.fori_loop(0, CPH, chunk_body, 0)

    return k(features, idx_chunks)


def _tc_dense(features, agg, w_top, w_bot):
    bn = 2000

    def body(f_ref, a_ref, wt_ref, wb_ref, o_ref):
        acc = jnp.dot(f_ref[...], wt_ref[...], preferred_element_type=jnp.float32)
        acc = acc + jnp.dot(a_ref[...], wb_ref[...], preferred_element_type=jnp.float32)
        o_ref[...] = jnp.maximum(acc, 0.0)

    return pl.pallas_call(
        body,
        grid=(N // bn,),
        in_specs=[
            pl.BlockSpec((bn, D), lambda i: (i, 0)),
            pl.BlockSpec((bn, D), lambda i: (i, 0)),
            pl.BlockSpec((D, D), lambda i: (0, 0)),
            pl.BlockSpec((D, D), lambda i: (0, 0)),
        ],
        out_specs=pl.BlockSpec((bn, D), lambda i: (i, 0)),
        out_shape=jax.ShapeDtypeStruct((N, D), jnp.float32),
    )(features, agg, w_top, w_bot)


def kernel(features, edge_look_up, kernel):
    idx = edge_look_up.astype(jnp.int32).reshape(-1)
    idx = jnp.pad(idx, (0, N_PAD * DEG - idx.shape[0]))
    feat_pad = jnp.pad(features, ((0, N_PAD - N), (0, 0)))
    agg = _sc_gather_mean(feat_pad, idx.reshape(N_PAD // C, EC))[:N]
    return _tc_dense(features, agg, kernel[:D], kernel[D:])
